# baseline (speedup 1.0000x reference)
import functools

import jax
import jax.numpy as jnp
from jax import lax
from jax.experimental import pallas as pl
from jax.experimental.pallas import tpu as pltpu
from jax.experimental.pallas import tpu_sc as plsc

N_NODES = 10000
N_EDGES = 320000
D = 128

NC = 2
NS = 16
NW = NC * NS
F_PER = D // NW
SEG = F_PER * N_NODES
CHUNK = 8000
NB = CHUNK // 16
NCHUNKS = N_EDGES // CHUNK


def _mm1_body(w_ref, x_ref, b_ref, o_ref):
    acc = lax.dot_general(w_ref[...], x_ref[...], (((1,), (1,)), ((), ())),
                          preferred_element_type=jnp.float32)
    o_ref[...] = jnp.maximum(acc + b_ref[...], 0.0)


def _mm2_body(a_ref, w_ref, b_ref, o_ref):
    acc = lax.dot_general(a_ref[...], w_ref[...], (((0,), (1,)), ((), ())),
                          preferred_element_type=jnp.float32)
    o_ref[...] = acc + b_ref[...]


def _segmax_body(h_hbm, s_hbm, r_hbm, out_hbm, h_v,
                 acc0, acc1, acc2, acc3, s_buf, r_buf):
    accs = (acc0, acc1, acc2, acc3)
    wid = lax.axis_index("s") * NC + lax.axis_index("c")
    base = wid * SEG
    pltpu.sync_copy(h_hbm.at[pl.ds(base, SEG)], h_v)

    def zero_body(i, c):
        for f in range(F_PER):
            accs[f][pl.ds(i * 16, 16)] = jnp.zeros((16,), jnp.float32)
        return c
    lax.fori_loop(0, N_NODES // 16, zero_body, 0)
    s_buf[pl.ds(CHUNK, 16)] = jnp.zeros((16,), jnp.int32)
    r_buf[pl.ds(CHUNK, 16)] = jnp.zeros((16,), jnp.int32)

    def chunk_body(g, c):
        pltpu.sync_copy(s_hbm.at[pl.ds(g * CHUNK, CHUNK)],
                        s_buf.at[pl.ds(0, CHUNK)])
        pltpu.sync_copy(r_hbm.at[pl.ds(g * CHUNK, CHUNK)],
                        r_buf.at[pl.ds(0, CHUNK)])

        def fast_batch(b, carry):
            s16, r16, hvs, unresolved = carry
            nb = (b + 1) * 16
            s_n = s_buf[pl.ds(nb, 16)]
            r_n = r_buf[pl.ds(nb, 16)]
            hv_n = tuple(plsc.load_gather(h_v, [s_n + (f * N_NODES)])
                         for f in range(F_PER))
            news = []
            for f in range(F_PER):
                cur = plsc.load_gather(accs[f], [r16])
                nw = jnp.maximum(cur, hvs[f])
                plsc.store_scatter(accs[f], [r16], nw)
                news.append(nw)
            pends = []
            for f in range(F_PER):
                chk = plsc.load_gather(accs[f], [r16])
                pends.append(chk < news[f])
            for f in range(F_PER):
                plsc.store_scatter(accs[f], [r16], news[f], mask=pends[f])
            for f in range(F_PER):
                chk = plsc.load_gather(accs[f], [r16])
                unresolved = unresolved | (chk < news[f])
            return (s_n, r_n, hv_n, unresolved)

        s0 = s_buf[pl.ds(0, 16)]
        r0 = r_buf[pl.ds(0, 16)]
        hv0 = tuple(plsc.load_gather(h_v, [s0 + (f * N_NODES)])
                    for f in range(F_PER))
        carry = (s0, r0, hv0, jnp.zeros((16,), jnp.bool_))
        carry = lax.fori_loop(0, NB, fast_batch, carry, unroll=2)
        unresolved = carry[3]

        @pl.when(jnp.any(unresolved))
        def _slow():
            def slow_batch(b, c2):
                s16 = s_buf[pl.ds(b * 16, 16)]
                r16 = r_buf[pl.ds(b * 16, 16)]
                news, pends = [], []
                for f in range(F_PER):
                    hv = plsc.load_gather(h_v, [s16 + (f * N_NODES)])
                    cur = plsc.load_gather(accs[f], [r16])
                    nw = jnp.maximum(cur, hv)
                    plsc.store_scatter(accs[f], [r16], nw)
                    chk = plsc.load_gather(accs[f], [r16])
                    news.append(nw)
                    pends.append(chk < nw)

                def cond(ps):
                    return jnp.any(ps[0] | ps[1] | ps[2] | ps[3])

                def retry(ps):
                    out_ps = []
                    for f in range(F_PER):
                        plsc.store_scatter(accs[f], [r16], news[f],
                                           mask=ps[f])
                        chk = plsc.load_gather(accs[f], [r16])
                        out_ps.append(ps[f] & (chk < news[f]))
                    return tuple(out_ps)

                lax.while_loop(cond, retry, tuple(pends))
                return c2
            lax.fori_loop(0, NB, slow_batch, 0)
        return c
    lax.fori_loop(0, NCHUNKS, chunk_body, 0)

    for f in range(F_PER):
        pltpu.sync_copy(accs[f], out_hbm.at[pl.ds(base + f * N_NODES, N_NODES)])


_segmax = functools.partial(
    pl.kernel,
    mesh=plsc.VectorSubcoreMesh(core_axis_name="c", subcore_axis_name="s",
                                num_cores=NC, num_subcores=NS),
    out_type=jax.ShapeDtypeStruct((D * N_NODES,), jnp.float32),
    compiler_params=pltpu.CompilerParams(needs_layout_passes=False),
    scratch_types=[
        pltpu.VMEM((SEG,), jnp.float32),
        pltpu.VMEM((N_NODES,), jnp.float32),
        pltpu.VMEM((N_NODES,), jnp.float32),
        pltpu.VMEM((N_NODES,), jnp.float32),
        pltpu.VMEM((N_NODES,), jnp.float32),
        pltpu.VMEM((CHUNK + 16,), jnp.int32),
        pltpu.VMEM((CHUNK + 16,), jnp.int32),
    ],
)(_segmax_body)


def kernel(node_features, senders, receivers, W1, b1, W2, b2):
    h_T = pl.pallas_call(
        _mm1_body,
        out_shape=jax.ShapeDtypeStruct((D, N_NODES), jnp.float32),
    )(W1, node_features, b1.reshape(D, 1))

    agg_flat = _segmax(h_T.reshape(-1),
                       senders.astype(jnp.int32),
                       receivers.astype(jnp.int32))

    nodes = pl.pallas_call(
        _mm2_body,
        out_shape=jax.ShapeDtypeStruct((N_NODES, D), jnp.float32),
    )(agg_flat.reshape(D, N_NODES), W2, b2.reshape(1, D))
    return nodes


# speedup vs baseline: 1.3998x; 1.3998x over previous
import functools

import jax
import jax.numpy as jnp
from jax import lax
from jax.experimental import pallas as pl
from jax.experimental.pallas import tpu as pltpu
from jax.experimental.pallas import tpu_sc as plsc

N_NODES = 10000
N_EDGES = 320000
D = 128

NC = 2
NS = 16
NW = NC * NS
F_PER = D // NW
SEG = F_PER * N_NODES
CHUNK = 8000
NB = CHUNK // 16
NCHUNKS = N_EDGES // CHUNK


def _mm1_body(w_ref, x_ref, b_ref, o_ref):
    acc = lax.dot_general(w_ref[...], x_ref[...], (((1,), (1,)), ((), ())),
                          preferred_element_type=jnp.float32)
    o_ref[...] = jnp.maximum(acc + b_ref[...], 0.0)


def _mm2_body(a_ref, w_ref, b_ref, o_ref):
    acc = lax.dot_general(a_ref[...], w_ref[...], (((0,), (1,)), ((), ())),
                          preferred_element_type=jnp.float32)
    o_ref[...] = acc + b_ref[...]


def _segmax_body(h_hbm, s_hbm, r_hbm, out_hbm, h_v,
                 acc0, acc1, acc2, acc3, s_buf, r_buf):
    accs = (acc0, acc1, acc2, acc3)
    wid = lax.axis_index("s") * NC + lax.axis_index("c")
    base = wid * SEG
    pltpu.sync_copy(h_hbm.at[pl.ds(base, SEG)], h_v)

    def zero_body(i, c):
        for f in range(F_PER):
            accs[f][pl.ds(i * 16, 16)] = jnp.zeros((16,), jnp.float32)
        return c
    lax.fori_loop(0, N_NODES // 16, zero_body, 0)
    s_buf[pl.ds(CHUNK, 16)] = jnp.zeros((16,), jnp.int32)
    r_buf[pl.ds(CHUNK, 16)] = jnp.zeros((16,), jnp.int32)

    def chunk_body(g, c):
        pltpu.sync_copy(s_hbm.at[pl.ds(g * CHUNK, CHUNK)],
                        s_buf.at[pl.ds(0, CHUNK)])
        pltpu.sync_copy(r_hbm.at[pl.ds(g * CHUNK, CHUNK)],
                        r_buf.at[pl.ds(0, CHUNK)])

        def fast_batch(b, carry):
            s16, r16, hvs, unresolved = carry
            nb = (b + 1) * 16
            s_n = s_buf[pl.ds(nb, 16)]
            r_n = r_buf[pl.ds(nb, 16)]
            hv_n = tuple(plsc.load_gather(h_v, [s_n + (f * N_NODES)])
                         for f in range(F_PER))
            curs = [plsc.load_gather(accs[f], [r16]) for f in range(F_PER)]
            news = [jnp.maximum(curs[f], hvs[f]) for f in range(F_PER)]
            for f in range(F_PER):
                plsc.store_scatter(accs[f], [r16], news[f])
            pends = []
            for f in range(F_PER):
                chk = plsc.load_gather(accs[f], [r16])
                pends.append(chk < news[f])
            for f in range(F_PER):
                plsc.store_scatter(accs[f], [r16], news[f], mask=pends[f])
            for f in range(F_PER):
                chk = plsc.load_gather(accs[f], [r16])
                unresolved = unresolved | (chk < news[f])
            return (s_n, r_n, hv_n, unresolved)

        s0 = s_buf[pl.ds(0, 16)]
        r0 = r_buf[pl.ds(0, 16)]
        hv0 = tuple(plsc.load_gather(h_v, [s0 + (f * N_NODES)])
                    for f in range(F_PER))
        carry = (s0, r0, hv0, jnp.zeros((16,), jnp.bool_))
        carry = lax.fori_loop(0, NB, fast_batch, carry, unroll=2)
        unresolved = carry[3]

        @pl.when(jnp.any(unresolved))
        def _slow():
            def slow_batch(b, c2):
                s16 = s_buf[pl.ds(b * 16, 16)]
                r16 = r_buf[pl.ds(b * 16, 16)]
                news, pends = [], []
                for f in range(F_PER):
                    hv = plsc.load_gather(h_v, [s16 + (f * N_NODES)])
                    cur = plsc.load_gather(accs[f], [r16])
                    nw = jnp.maximum(cur, hv)
                    plsc.store_scatter(accs[f], [r16], nw)
                    chk = plsc.load_gather(accs[f], [r16])
                    news.append(nw)
                    pends.append(chk < nw)

                def cond(ps):
                    return jnp.any(ps[0] | ps[1] | ps[2] | ps[3])

                def retry(ps):
                    out_ps = []
                    for f in range(F_PER):
                        plsc.store_scatter(accs[f], [r16], news[f],
                                           mask=ps[f])
                        chk = plsc.load_gather(accs[f], [r16])
                        out_ps.append(ps[f] & (chk < news[f]))
                    return tuple(out_ps)

                lax.while_loop(cond, retry, tuple(pends))
                return c2
            lax.fori_loop(0, NB, slow_batch, 0)
        return c
    lax.fori_loop(0, NCHUNKS, chunk_body, 0)

    for f in range(F_PER):
        pltpu.sync_copy(accs[f], out_hbm.at[pl.ds(base + f * N_NODES, N_NODES)])


_segmax = functools.partial(
    pl.kernel,
    mesh=plsc.VectorSubcoreMesh(core_axis_name="c", subcore_axis_name="s",
                                num_cores=NC, num_subcores=NS),
    out_type=jax.ShapeDtypeStruct((D * N_NODES,), jnp.float32),
    compiler_params=pltpu.CompilerParams(needs_layout_passes=False),
    scratch_types=[
        pltpu.VMEM((SEG,), jnp.float32),
        pltpu.VMEM((N_NODES,), jnp.float32),
        pltpu.VMEM((N_NODES,), jnp.float32),
        pltpu.VMEM((N_NODES,), jnp.float32),
        pltpu.VMEM((N_NODES,), jnp.float32),
        pltpu.VMEM((CHUNK + 16,), jnp.int32),
        pltpu.VMEM((CHUNK + 16,), jnp.int32),
    ],
)(_segmax_body)


def kernel(node_features, senders, receivers, W1, b1, W2, b2):
    h_T = pl.pallas_call(
        _mm1_body,
        out_shape=jax.ShapeDtypeStruct((D, N_NODES), jnp.float32),
    )(W1, node_features, b1.reshape(D, 1))

    agg_flat = _segmax(h_T.reshape(-1),
                       senders.astype(jnp.int32),
                       receivers.astype(jnp.int32))

    nodes = pl.pallas_call(
        _mm2_body,
        out_shape=jax.ShapeDtypeStruct((N_NODES, D), jnp.float32),
    )(agg_flat.reshape(D, N_NODES), W2, b2.reshape(1, D))
    return nodes


# speedup vs baseline: 1.4791x; 1.0567x over previous
import functools

import jax
import jax.numpy as jnp
from jax import lax
from jax.experimental import pallas as pl
from jax.experimental.pallas import tpu as pltpu
from jax.experimental.pallas import tpu_sc as plsc

N_NODES = 10000
N_EDGES = 320000
D = 128

NC = 2
NS = 16
NW = NC * NS
F_PER = D // NW
SEG = F_PER * N_NODES
CHUNK = 8000
NB = CHUNK // 16
NCHUNKS = N_EDGES // CHUNK


def _mm1_body(w_ref, x_ref, b_ref, o_ref):
    acc = lax.dot_general(w_ref[...], x_ref[...], (((1,), (1,)), ((), ())),
                          preferred_element_type=jnp.float32)
    o_ref[...] = jnp.maximum(acc + b_ref[...], 0.0)


def _mm2_body(a_ref, w_ref, b_ref, o_ref):
    acc = lax.dot_general(a_ref[...], w_ref[...], (((0,), (1,)), ((), ())),
                          preferred_element_type=jnp.float32)
    o_ref[...] = acc + b_ref[...]


def _segmax_body(h_hbm, s_hbm, r_hbm, out_hbm, h_v,
                 acc0, acc1, acc2, acc3, s_buf, r_buf):
    accs = (acc0, acc1, acc2, acc3)
    wid = lax.axis_index("s") * NC + lax.axis_index("c")
    base = wid * SEG
    pltpu.sync_copy(h_hbm.at[pl.ds(base, SEG)], h_v)

    lanes = lax.iota(jnp.int32, 16)
    probe = jnp.where(lanes < 3, 0, lanes)
    counts_probe, _ = plsc.scan_count(probe)
    thresh3 = jnp.max(counts_probe)

    def zero_body(i, c):
        for f in range(F_PER):
            accs[f][pl.ds(i * 16, 16)] = jnp.zeros((16,), jnp.float32)
        return c
    lax.fori_loop(0, N_NODES // 16, zero_body, 0)
    s_buf[pl.ds(CHUNK, 16)] = jnp.zeros((16,), jnp.int32)
    r_buf[pl.ds(CHUNK, 16)] = jnp.zeros((16,), jnp.int32)

    def chunk_body(g, c):
        pltpu.sync_copy(s_hbm.at[pl.ds(g * CHUNK, CHUNK)],
                        s_buf.at[pl.ds(0, CHUNK)])
        pltpu.sync_copy(r_hbm.at[pl.ds(g * CHUNK, CHUNK)],
                        r_buf.at[pl.ds(0, CHUNK)])

        def fast_batch(b, carry):
            s16, r16, hvs, unresolved = carry
            nb = (b + 1) * 16
            s_n = s_buf[pl.ds(nb, 16)]
            r_n = r_buf[pl.ds(nb, 16)]
            hv_n = tuple(plsc.load_gather(h_v, [s_n + (f * N_NODES)])
                         for f in range(F_PER))
            curs = [plsc.load_gather(accs[f], [r16]) for f in range(F_PER)]
            news = [jnp.maximum(curs[f], hvs[f]) for f in range(F_PER)]
            for f in range(F_PER):
                plsc.store_scatter(accs[f], [r16], news[f])
            pends = []
            for f in range(F_PER):
                chk = plsc.load_gather(accs[f], [r16])
                pends.append(chk < news[f])
            for f in range(F_PER):
                plsc.store_scatter(accs[f], [r16], news[f], mask=pends[f])
            counts, _ = plsc.scan_count(r16)
            unresolved = unresolved | (counts >= thresh3)
            return (s_n, r_n, hv_n, unresolved)

        s0 = s_buf[pl.ds(0, 16)]
        r0 = r_buf[pl.ds(0, 16)]
        hv0 = tuple(plsc.load_gather(h_v, [s0 + (f * N_NODES)])
                    for f in range(F_PER))
        carry = (s0, r0, hv0, jnp.zeros((16,), jnp.bool_))
        carry = lax.fori_loop(0, NB, fast_batch, carry, unroll=2)
        unresolved = carry[3]

        @pl.when(jnp.any(unresolved))
        def _slow():
            def slow_batch(b, c2):
                s16 = s_buf[pl.ds(b * 16, 16)]
                r16 = r_buf[pl.ds(b * 16, 16)]
                news, pends = [], []
                for f in range(F_PER):
                    hv = plsc.load_gather(h_v, [s16 + (f * N_NODES)])
                    cur = plsc.load_gather(accs[f], [r16])
                    nw = jnp.maximum(cur, hv)
                    plsc.store_scatter(accs[f], [r16], nw)
                    chk = plsc.load_gather(accs[f], [r16])
                    news.append(nw)
                    pends.append(chk < nw)

                def cond(ps):
                    return jnp.any(ps[0] | ps[1] | ps[2] | ps[3])

                def retry(ps):
                    out_ps = []
                    for f in range(F_PER):
                        plsc.store_scatter(accs[f], [r16], news[f],
                                           mask=ps[f])
                        chk = plsc.load_gather(accs[f], [r16])
                        out_ps.append(ps[f] & (chk < news[f]))
                    return tuple(out_ps)

                lax.while_loop(cond, retry, tuple(pends))
                return c2
            lax.fori_loop(0, NB, slow_batch, 0)
        return c
    lax.fori_loop(0, NCHUNKS, chunk_body, 0)

    for f in range(F_PER):
        pltpu.sync_copy(accs[f], out_hbm.at[pl.ds(base + f * N_NODES, N_NODES)])


_segmax = functools.partial(
    pl.kernel,
    mesh=plsc.VectorSubcoreMesh(core_axis_name="c", subcore_axis_name="s",
                                num_cores=NC, num_subcores=NS),
    out_type=jax.ShapeDtypeStruct((D * N_NODES,), jnp.float32),
    compiler_params=pltpu.CompilerParams(needs_layout_passes=False),
    scratch_types=[
        pltpu.VMEM((SEG,), jnp.float32),
        pltpu.VMEM((N_NODES,), jnp.float32),
        pltpu.VMEM((N_NODES,), jnp.float32),
        pltpu.VMEM((N_NODES,), jnp.float32),
        pltpu.VMEM((N_NODES,), jnp.float32),
        pltpu.VMEM((CHUNK + 16,), jnp.int32),
        pltpu.VMEM((CHUNK + 16,), jnp.int32),
    ],
)(_segmax_body)


def kernel(node_features, senders, receivers, W1, b1, W2, b2):
    h_T = pl.pallas_call(
        _mm1_body,
        out_shape=jax.ShapeDtypeStruct((D, N_NODES), jnp.float32),
    )(W1, node_features, b1.reshape(D, 1))

    agg_flat = _segmax(h_T.reshape(-1),
                       senders.astype(jnp.int32),
                       receivers.astype(jnp.int32))

    nodes = pl.pallas_call(
        _mm2_body,
        out_shape=jax.ShapeDtypeStruct((N_NODES, D), jnp.float32),
    )(agg_flat.reshape(D, N_NODES), W2, b2.reshape(1, D))
    return nodes


# speedup vs baseline: 1.6773x; 1.1340x over previous
import functools

import jax
import jax.numpy as jnp
from jax import lax
from jax.experimental import pallas as pl
from jax.experimental.pallas import tpu as pltpu
from jax.experimental.pallas import tpu_sc as plsc

N_NODES = 10000
N_EDGES = 320000
D = 128

NC = 2
NS = 16
NW = NC * NS
F_PER = D // NW
SEG = F_PER * N_NODES
CHUNK = 8000
NB = CHUNK // 16
NCHUNKS = N_EDGES // CHUNK
SLOT = CHUNK + 16


def _mm1_body(w_ref, x_ref, b_ref, o_ref):
    acc = lax.dot_general(w_ref[...], x_ref[...], (((1,), (1,)), ((), ())),
                          preferred_element_type=jnp.float32)
    o_ref[...] = jnp.maximum(acc + b_ref[...], 0.0)


def _mm2_body(a_ref, w_ref, b_ref, o_ref):
    acc = lax.dot_general(a_ref[...], w_ref[...], (((0,), (1,)), ((), ())),
                          preferred_element_type=jnp.float32)
    o_ref[...] = acc + b_ref[...]


def _segmax_body(h_hbm, s_hbm, r_hbm, out_hbm, h_v,
                 acc0, acc1, acc2, acc3, s_buf, r_buf,
                 sem_s0, sem_s1, sem_r0, sem_r1):
    accs = (acc0, acc1, acc2, acc3)
    sem_s = (sem_s0, sem_s1)
    sem_r = (sem_r0, sem_r1)
    wid = lax.axis_index("s") * NC + lax.axis_index("c")
    base = wid * SEG
    pltpu.sync_copy(h_hbm.at[pl.ds(base, SEG)], h_v)

    lanes = lax.iota(jnp.int32, 16)
    probe = jnp.where(lanes < 3, 0, lanes)
    counts_probe, _ = plsc.scan_count(probe)
    thresh3 = jnp.max(counts_probe)

    def zero_body(i, c):
        for f in range(F_PER):
            accs[f][pl.ds(i * 16, 16)] = jnp.zeros((16,), jnp.float32)
        return c
    lax.fori_loop(0, N_NODES // 16, zero_body, 0)
    for so in (0, SLOT):
        s_buf[pl.ds(so + CHUNK, 16)] = jnp.zeros((16,), jnp.int32)
        r_buf[pl.ds(so + CHUNK, 16)] = jnp.zeros((16,), jnp.int32)

    def start_chunk(g, slot):
        pltpu.async_copy(s_hbm.at[pl.ds(g * CHUNK, CHUNK)],
                         s_buf.at[pl.ds(slot * SLOT, CHUNK)], sem_s[slot])
        pltpu.async_copy(r_hbm.at[pl.ds(g * CHUNK, CHUNK)],
                         r_buf.at[pl.ds(slot * SLOT, CHUNK)], sem_r[slot])

    def wait_chunk(g, slot):
        pltpu.make_async_copy(
            s_hbm.at[pl.ds(g * CHUNK, CHUNK)],
            s_buf.at[pl.ds(slot * SLOT, CHUNK)], sem_s[slot]).wait()
        pltpu.make_async_copy(
            r_hbm.at[pl.ds(g * CHUNK, CHUNK)],
            r_buf.at[pl.ds(slot * SLOT, CHUNK)], sem_r[slot]).wait()

    def process_chunk(so):
        def fast_batch(b, carry):
            s16, r16, hvs, unresolved = carry
            nb = so + (b + 1) * 16
            s_n = s_buf[pl.ds(nb, 16)]
            r_n = r_buf[pl.ds(nb, 16)]
            hv_n = tuple(plsc.load_gather(h_v, [s_n + (f * N_NODES)])
                         for f in range(F_PER))
            curs = [plsc.load_gather(accs[f], [r16]) for f in range(F_PER)]
            news = [jnp.maximum(curs[f], hvs[f]) for f in range(F_PER)]
            for f in range(F_PER):
                plsc.store_scatter(accs[f], [r16], news[f])
            pends = []
            for f in range(F_PER):
                chk = plsc.load_gather(accs[f], [r16])
                pends.append(chk < news[f])
            for f in range(F_PER):
                plsc.store_scatter(accs[f], [r16], news[f], mask=pends[f])
            counts, _ = plsc.scan_count(r16)
            unresolved = unresolved | (counts >= thresh3)
            return (s_n, r_n, hv_n, unresolved)

        s0 = s_buf[pl.ds(so, 16)]
        r0 = r_buf[pl.ds(so, 16)]
        hv0 = tuple(plsc.load_gather(h_v, [s0 + (f * N_NODES)])
                    for f in range(F_PER))
        carry = (s0, r0, hv0, jnp.zeros((16,), jnp.bool_))
        carry = lax.fori_loop(0, NB, fast_batch, carry, unroll=2)
        unresolved = carry[3]

        @pl.when(jnp.any(unresolved))
        def _slow():
            def slow_batch(b, c2):
                s16 = s_buf[pl.ds(so + b * 16, 16)]
                r16 = r_buf[pl.ds(so + b * 16, 16)]
                news, pends = [], []
                for f in range(F_PER):
                    hv = plsc.load_gather(h_v, [s16 + (f * N_NODES)])
                    cur = plsc.load_gather(accs[f], [r16])
                    nw = jnp.maximum(cur, hv)
                    plsc.store_scatter(accs[f], [r16], nw)
                    chk = plsc.load_gather(accs[f], [r16])
                    news.append(nw)
                    pends.append(chk < nw)

                def cond(ps):
                    return jnp.any(ps[0] | ps[1] | ps[2] | ps[3])

                def retry(ps):
                    out_ps = []
                    for f in range(F_PER):
                        plsc.store_scatter(accs[f], [r16], news[f],
                                           mask=ps[f])
                        chk = plsc.load_gather(accs[f], [r16])
                        out_ps.append(ps[f] & (chk < news[f]))
                    return tuple(out_ps)

                lax.while_loop(cond, retry, tuple(pends))
                return c2
            lax.fori_loop(0, NB, slow_batch, 0)

    start_chunk(0, 0)
    start_chunk(1, 1)

    def super_body(gs, c):
        for slot in (0, 1):
            g = gs * 2 + slot
            wait_chunk(g, slot)
            process_chunk(slot * SLOT)

            @pl.when(g + 2 < NCHUNKS)
            def _():
                start_chunk(g + 2, slot)
        return c
    lax.fori_loop(0, NCHUNKS // 2, super_body, 0)

    for f in range(F_PER):
        pltpu.sync_copy(accs[f], out_hbm.at[pl.ds(base + f * N_NODES, N_NODES)])


_segmax = functools.partial(
    pl.kernel,
    mesh=plsc.VectorSubcoreMesh(core_axis_name="c", subcore_axis_name="s",
                                num_cores=NC, num_subcores=NS),
    out_type=jax.ShapeDtypeStruct((D * N_NODES,), jnp.float32),
    compiler_params=pltpu.CompilerParams(needs_layout_passes=False),
    scratch_types=[
        pltpu.VMEM((SEG,), jnp.float32),
        pltpu.VMEM((N_NODES,), jnp.float32),
        pltpu.VMEM((N_NODES,), jnp.float32),
        pltpu.VMEM((N_NODES,), jnp.float32),
        pltpu.VMEM((N_NODES,), jnp.float32),
        pltpu.VMEM((2 * (CHUNK + 16),), jnp.int32),
        pltpu.VMEM((2 * (CHUNK + 16),), jnp.int32),
        pltpu.SemaphoreType.DMA,
        pltpu.SemaphoreType.DMA,
        pltpu.SemaphoreType.DMA,
        pltpu.SemaphoreType.DMA,
    ],
)(_segmax_body)


def kernel(node_features, senders, receivers, W1, b1, W2, b2):
    h_T = pl.pallas_call(
        _mm1_body,
        out_shape=jax.ShapeDtypeStruct((D, N_NODES), jnp.float32),
    )(W1, node_features, b1.reshape(D, 1))

    agg_flat = _segmax(h_T.reshape(-1),
                       senders.astype(jnp.int32),
                       receivers.astype(jnp.int32))

    nodes = pl.pallas_call(
        _mm2_body,
        out_shape=jax.ShapeDtypeStruct((N_NODES, D), jnp.float32),
    )(agg_flat.reshape(D, N_NODES), W2, b2.reshape(1, D))
    return nodes


# speedup vs baseline: 1.7097x; 1.0193x over previous
import functools

import jax
import jax.numpy as jnp
from jax import lax
from jax.experimental import pallas as pl
from jax.experimental.pallas import tpu as pltpu
from jax.experimental.pallas import tpu_sc as plsc

N_NODES = 10000
N_EDGES = 320000
D = 128

NC = 2
NS = 16
NW = NC * NS
F_PER = D // NW
SEG = F_PER * N_NODES
CHUNK = 1600
NB = CHUNK // 16
NB2 = NB // 2
NCHUNKS = N_EDGES // CHUNK
SLOT = CHUNK + 32


def _mm1_body(w_ref, x_ref, b_ref, o_ref):
    acc = lax.dot_general(w_ref[...], x_ref[...], (((1,), (1,)), ((), ())),
                          preferred_element_type=jnp.float32)
    o_ref[...] = jnp.maximum(acc + b_ref[...], 0.0)


def _mm2_body(a_ref, w_ref, b_ref, o_ref):
    acc = lax.dot_general(a_ref[...], w_ref[...], (((0,), (1,)), ((), ())),
                          preferred_element_type=jnp.float32)
    o_ref[...] = acc + b_ref[...]


def _segmax_body(h_hbm, s_hbm, r_hbm, out_hbm, h_v,
                 accA0, accA1, accA2, accA3, accB0, accB1, accB2, accB3,
                 s_buf, r_buf, sem_s0, sem_s1, sem_r0, sem_r1):
    accsA = (accA0, accA1, accA2, accA3)
    accsB = (accB0, accB1, accB2, accB3)
    sem_s = (sem_s0, sem_s1)
    sem_r = (sem_r0, sem_r1)
    wid = lax.axis_index("s") * NC + lax.axis_index("c")
    base = wid * SEG
    pltpu.sync_copy(h_hbm.at[pl.ds(base, SEG)], h_v)

    lanes = lax.iota(jnp.int32, 16)
    probe = jnp.where(lanes < 3, 0, lanes)
    counts_probe, _ = plsc.scan_count(probe)
    thresh3 = jnp.max(counts_probe)

    def zero_body(i, c):
        for f in range(F_PER):
            accsA[f][pl.ds(i * 16, 16)] = jnp.zeros((16,), jnp.float32)
            accsB[f][pl.ds(i * 16, 16)] = jnp.zeros((16,), jnp.float32)
        return c
    lax.fori_loop(0, N_NODES // 16, zero_body, 0)
    for so in (0, SLOT):
        for t in (0, 16):
            s_buf[pl.ds(so + CHUNK + t, 16)] = jnp.zeros((16,), jnp.int32)
            r_buf[pl.ds(so + CHUNK + t, 16)] = jnp.zeros((16,), jnp.int32)

    def start_chunk(g, slot):
        pltpu.async_copy(s_hbm.at[pl.ds(g * CHUNK, CHUNK)],
                         s_buf.at[pl.ds(slot * SLOT, CHUNK)], sem_s[slot])
        pltpu.async_copy(r_hbm.at[pl.ds(g * CHUNK, CHUNK)],
                         r_buf.at[pl.ds(slot * SLOT, CHUNK)], sem_r[slot])

    def wait_chunk(g, slot):
        pltpu.make_async_copy(
            s_hbm.at[pl.ds(g * CHUNK, CHUNK)],
            s_buf.at[pl.ds(slot * SLOT, CHUNK)], sem_s[slot]).wait()
        pltpu.make_async_copy(
            r_hbm.at[pl.ds(g * CHUNK, CHUNK)],
            r_buf.at[pl.ds(slot * SLOT, CHUNK)], sem_r[slot]).wait()

    def rmw_batch(accs, r16, hvs, unresolved):
        curs = [plsc.load_gather(accs[f], [r16]) for f in range(F_PER)]
        news = [jnp.maximum(curs[f], hvs[f]) for f in range(F_PER)]
        for f in range(F_PER):
            plsc.store_scatter(accs[f], [r16], news[f])
        pends = []
        for f in range(F_PER):
            chk = plsc.load_gather(accs[f], [r16])
            pends.append(chk < news[f])
        for f in range(F_PER):
            plsc.store_scatter(accs[f], [r16], news[f], mask=pends[f])
        counts, _ = plsc.scan_count(r16)
        return unresolved | (counts >= thresh3)

    def process_chunk(so):
        def fast_pair(b2, carry):
            sA, rA, hvA, sB, rB, hvB, unresolved = carry
            nb = so + (b2 + 1) * 32
            sA_n = s_buf[pl.ds(nb, 16)]
            rA_n = r_buf[pl.ds(nb, 16)]
            sB_n = s_buf[pl.ds(nb + 16, 16)]
            rB_n = r_buf[pl.ds(nb + 16, 16)]
            hvA_n = tuple(plsc.load_gather(h_v, [sA_n + (f * N_NODES)])
                          for f in range(F_PER))
            hvB_n = tuple(plsc.load_gather(h_v, [sB_n + (f * N_NODES)])
                          for f in range(F_PER))
            unresolved = rmw_batch(accsA, rA, hvA, unresolved)
            unresolved = rmw_batch(accsB, rB, hvB, unresolved)
            return (sA_n, rA_n, hvA_n, sB_n, rB_n, hvB_n, unresolved)

        sA0 = s_buf[pl.ds(so, 16)]
        rA0 = r_buf[pl.ds(so, 16)]
        sB0 = s_buf[pl.ds(so + 16, 16)]
        rB0 = r_buf[pl.ds(so + 16, 16)]
        hvA0 = tuple(plsc.load_gather(h_v, [sA0 + (f * N_NODES)])
                     for f in range(F_PER))
        hvB0 = tuple(plsc.load_gather(h_v, [sB0 + (f * N_NODES)])
                     for f in range(F_PER))
        carry = (sA0, rA0, hvA0, sB0, rB0, hvB0,
                 jnp.zeros((16,), jnp.bool_))
        carry = lax.fori_loop(0, NB2, fast_pair, carry)
        unresolved = carry[6]

        @pl.when(jnp.any(unresolved))
        def _slow():
            for parity, accs in ((0, accsA), (1, accsB)):
                def slow_batch(b2, c2, parity=parity, accs=accs):
                    off = so + b2 * 32 + parity * 16
                    s16 = s_buf[pl.ds(off, 16)]
                    r16 = r_buf[pl.ds(off, 16)]
                    news, pends = [], []
                    for f in range(F_PER):
                        hv = plsc.load_gather(h_v, [s16 + (f * N_NODES)])
                        cur = plsc.load_gather(accs[f], [r16])
                        nw = jnp.maximum(cur, hv)
                        plsc.store_scatter(accs[f], [r16], nw)
                        chk = plsc.load_gather(accs[f], [r16])
                        news.append(nw)
                        pends.append(chk < nw)

                    def cond(ps):
                        return jnp.any(ps[0] | ps[1] | ps[2] | ps[3])

                    def retry(ps):
                        out_ps = []
                        for f in range(F_PER):
                            plsc.store_scatter(accs[f], [r16], news[f],
                                               mask=ps[f])
                            chk = plsc.load_gather(accs[f], [r16])
                            out_ps.append(ps[f] & (chk < news[f]))
                        return tuple(out_ps)

                    lax.while_loop(cond, retry, tuple(pends))
                    return c2
                lax.fori_loop(0, NB2, slow_batch, 0)

    start_chunk(0, 0)
    start_chunk(1, 1)

    def super_body(gs, c):
        for slot in (0, 1):
            g = gs * 2 + slot
            wait_chunk(g, slot)
            process_chunk(slot * SLOT)

            @pl.when(g + 2 < NCHUNKS)
            def _():
                start_chunk(g + 2, slot)
        return c
    lax.fori_loop(0, NCHUNKS // 2, super_body, 0)

    def merge_body(i, c):
        for f in range(F_PER):
            a = accsA[f][pl.ds(i * 16, 16)]
            b = accsB[f][pl.ds(i * 16, 16)]
            accsA[f][pl.ds(i * 16, 16)] = jnp.maximum(a, b)
        return c
    lax.fori_loop(0, N_NODES // 16, merge_body, 0)

    for f in range(F_PER):
        pltpu.sync_copy(accsA[f],
                        out_hbm.at[pl.ds(base + f * N_NODES, N_NODES)])


_segmax = functools.partial(
    pl.kernel,
    mesh=plsc.VectorSubcoreMesh(core_axis_name="c", subcore_axis_name="s",
                                num_cores=NC, num_subcores=NS),
    out_type=jax.ShapeDtypeStruct((D * N_NODES,), jnp.float32),
    compiler_params=pltpu.CompilerParams(needs_layout_passes=False),
    scratch_types=[
        pltpu.VMEM((SEG,), jnp.float32),
        pltpu.VMEM((N_NODES,), jnp.float32),
        pltpu.VMEM((N_NODES,), jnp.float32),
        pltpu.VMEM((N_NODES,), jnp.float32),
        pltpu.VMEM((N_NODES,), jnp.float32),
        pltpu.VMEM((N_NODES,), jnp.float32),
        pltpu.VMEM((N_NODES,), jnp.float32),
        pltpu.VMEM((N_NODES,), jnp.float32),
        pltpu.VMEM((N_NODES,), jnp.float32),
        pltpu.VMEM((2 * SLOT,), jnp.int32),
        pltpu.VMEM((2 * SLOT,), jnp.int32),
        pltpu.SemaphoreType.DMA,
        pltpu.SemaphoreType.DMA,
        pltpu.SemaphoreType.DMA,
        pltpu.SemaphoreType.DMA,
    ],
)(_segmax_body)


def kernel(node_features, senders, receivers, W1, b1, W2, b2):
    h_T = pl.pallas_call(
        _mm1_body,
        out_shape=jax.ShapeDtypeStruct((D, N_NODES), jnp.float32),
    )(W1, node_features, b1.reshape(D, 1))

    agg_flat = _segmax(h_T.reshape(-1),
                       senders.astype(jnp.int32),
                       receivers.astype(jnp.int32))

    nodes = pl.pallas_call(
        _mm2_body,
        out_shape=jax.ShapeDtypeStruct((N_NODES, D), jnp.float32),
    )(agg_flat.reshape(D, N_NODES), W2, b2.reshape(1, D))
    return nodes


# speedup vs baseline: 1.7343x; 1.0144x over previous
import functools

import jax
import jax.numpy as jnp
from jax import lax
from jax.experimental import pallas as pl
from jax.experimental.pallas import tpu as pltpu
from jax.experimental.pallas import tpu_sc as plsc

N_NODES = 10000
N_EDGES = 320000
D = 128

NC = 2
NS = 16
NW = NC * NS
F_PER = D // NW
SEG = F_PER * N_NODES
CHUNK = 1600
NB = CHUNK // 16
NB2 = NB // 2
NCHUNKS = N_EDGES // CHUNK
SLOT = CHUNK + 32


def _mm1_body(w_ref, x_ref, b_ref, o_ref):
    acc = lax.dot_general(w_ref[...], x_ref[...], (((1,), (1,)), ((), ())),
                          preferred_element_type=jnp.float32)
    o_ref[...] = jnp.maximum(acc + b_ref[...], 0.0)


def _mm2_body(a_ref, w_ref, b_ref, o_ref):
    acc = lax.dot_general(a_ref[...], w_ref[...], (((0,), (1,)), ((), ())),
                          preferred_element_type=jnp.float32)
    o_ref[...] = acc + b_ref[...]


def _segmax_body(h_hbm, s_hbm, r_hbm, out_hbm, h_v,
                 accA0, accA1, accA2, accA3, accB0, accB1, accB2, accB3,
                 s_buf, r_buf, sem_s0, sem_s1, sem_r0, sem_r1):
    accsA = (accA0, accA1, accA2, accA3)
    accsB = (accB0, accB1, accB2, accB3)
    sem_s = (sem_s0, sem_s1)
    sem_r = (sem_r0, sem_r1)
    wid = lax.axis_index("s") * NC + lax.axis_index("c")
    base = wid * SEG
    pltpu.sync_copy(h_hbm.at[pl.ds(base, SEG)], h_v)

    lanes = lax.iota(jnp.int32, 16)
    probe = jnp.where(lanes < 3, 0, lanes)
    counts_probe, _ = plsc.scan_count(probe)
    thresh3 = jnp.max(counts_probe)

    def zero_body(i, c):
        for f in range(F_PER):
            accsA[f][pl.ds(i * 16, 16)] = jnp.zeros((16,), jnp.float32)
            accsB[f][pl.ds(i * 16, 16)] = jnp.zeros((16,), jnp.float32)
        return c
    lax.fori_loop(0, N_NODES // 16, zero_body, 0)
    for so in (0, SLOT):
        for t in (0, 16):
            s_buf[pl.ds(so + CHUNK + t, 16)] = jnp.zeros((16,), jnp.int32)
            r_buf[pl.ds(so + CHUNK + t, 16)] = jnp.zeros((16,), jnp.int32)

    def start_chunk(g, slot):
        pltpu.async_copy(s_hbm.at[pl.ds(g * CHUNK, CHUNK)],
                         s_buf.at[pl.ds(slot * SLOT, CHUNK)], sem_s[slot])
        pltpu.async_copy(r_hbm.at[pl.ds(g * CHUNK, CHUNK)],
                         r_buf.at[pl.ds(slot * SLOT, CHUNK)], sem_r[slot])

    def wait_chunk(g, slot):
        pltpu.make_async_copy(
            s_hbm.at[pl.ds(g * CHUNK, CHUNK)],
            s_buf.at[pl.ds(slot * SLOT, CHUNK)], sem_s[slot]).wait()
        pltpu.make_async_copy(
            r_hbm.at[pl.ds(g * CHUNK, CHUNK)],
            r_buf.at[pl.ds(slot * SLOT, CHUNK)], sem_r[slot]).wait()

    def rmw_batch(accs, r16, hvs, unresolved):
        curs = [plsc.load_gather(accs[f], [r16]) for f in range(F_PER)]
        news = [jnp.maximum(curs[f], hvs[f]) for f in range(F_PER)]
        for f in range(F_PER):
            plsc.store_scatter(accs[f], [r16], news[f])
        pends = []
        for f in range(F_PER):
            chk = plsc.load_gather(accs[f], [r16])
            pends.append(chk < news[f])
        for f in range(F_PER):
            plsc.store_scatter(accs[f], [r16], news[f], mask=pends[f])
        counts, _ = plsc.scan_count(r16)
        return unresolved | (counts >= thresh3)

    def process_chunk(so):
        def fast_pair(b2, carry):
            sA, rA, hvA, sB, rB, hvB, unresolved = carry
            nb = so + (b2 + 1) * 32
            sA_n = s_buf[pl.ds(nb, 16)]
            rA_n = r_buf[pl.ds(nb, 16)]
            sB_n = s_buf[pl.ds(nb + 16, 16)]
            rB_n = r_buf[pl.ds(nb + 16, 16)]
            hvA_n = tuple(plsc.load_gather(h_v, [sA_n + (f * N_NODES)])
                          for f in range(F_PER))
            hvB_n = tuple(plsc.load_gather(h_v, [sB_n + (f * N_NODES)])
                          for f in range(F_PER))
            unresolved = rmw_batch(accsA, rA, hvA, unresolved)
            unresolved = rmw_batch(accsB, rB, hvB, unresolved)
            return (sA_n, rA_n, hvA_n, sB_n, rB_n, hvB_n, unresolved)

        sA0 = s_buf[pl.ds(so, 16)]
        rA0 = r_buf[pl.ds(so, 16)]
        sB0 = s_buf[pl.ds(so + 16, 16)]
        rB0 = r_buf[pl.ds(so + 16, 16)]
        hvA0 = tuple(plsc.load_gather(h_v, [sA0 + (f * N_NODES)])
                     for f in range(F_PER))
        hvB0 = tuple(plsc.load_gather(h_v, [sB0 + (f * N_NODES)])
                     for f in range(F_PER))
        carry = (sA0, rA0, hvA0, sB0, rB0, hvB0,
                 jnp.zeros((16,), jnp.bool_))
        carry = lax.fori_loop(0, NB2, fast_pair, carry, unroll=2)
        unresolved = carry[6]

        @pl.when(jnp.any(unresolved))
        def _slow():
            for parity, accs in ((0, accsA), (1, accsB)):
                def slow_batch(b2, c2, parity=parity, accs=accs):
                    off = so + b2 * 32 + parity * 16
                    s16 = s_buf[pl.ds(off, 16)]
                    r16 = r_buf[pl.ds(off, 16)]
                    news, pends = [], []
                    for f in range(F_PER):
                        hv = plsc.load_gather(h_v, [s16 + (f * N_NODES)])
                        cur = plsc.load_gather(accs[f], [r16])
                        nw = jnp.maximum(cur, hv)
                        plsc.store_scatter(accs[f], [r16], nw)
                        chk = plsc.load_gather(accs[f], [r16])
                        news.append(nw)
                        pends.append(chk < nw)

                    def cond(ps):
                        return jnp.any(ps[0] | ps[1] | ps[2] | ps[3])

                    def retry(ps):
                        out_ps = []
                        for f in range(F_PER):
                            plsc.store_scatter(accs[f], [r16], news[f],
                                               mask=ps[f])
                            chk = plsc.load_gather(accs[f], [r16])
                            out_ps.append(ps[f] & (chk < news[f]))
                        return tuple(out_ps)

                    lax.while_loop(cond, retry, tuple(pends))
                    return c2
                lax.fori_loop(0, NB2, slow_batch, 0)

    start_chunk(0, 0)
    start_chunk(1, 1)

    def super_body(gs, c):
        for slot in (0, 1):
            g = gs * 2 + slot
            wait_chunk(g, slot)
            process_chunk(slot * SLOT)

            @pl.when(g + 2 < NCHUNKS)
            def _():
                start_chunk(g + 2, slot)
        return c
    lax.fori_loop(0, NCHUNKS // 2, super_body, 0)

    def merge_body(i, c):
        for f in range(F_PER):
            a = accsA[f][pl.ds(i * 16, 16)]
            b = accsB[f][pl.ds(i * 16, 16)]
            accsA[f][pl.ds(i * 16, 16)] = jnp.maximum(a, b)
        return c
    lax.fori_loop(0, N_NODES // 16, merge_body, 0)

    for f in range(F_PER):
        pltpu.sync_copy(accsA[f],
                        out_hbm.at[pl.ds(base + f * N_NODES, N_NODES)])


_segmax = functools.partial(
    pl.kernel,
    mesh=plsc.VectorSubcoreMesh(core_axis_name="c", subcore_axis_name="s",
                                num_cores=NC, num_subcores=NS),
    out_type=jax.ShapeDtypeStruct((D * N_NODES,), jnp.float32),
    compiler_params=pltpu.CompilerParams(needs_layout_passes=False),
    scratch_types=[
        pltpu.VMEM((SEG,), jnp.float32),
        pltpu.VMEM((N_NODES,), jnp.float32),
        pltpu.VMEM((N_NODES,), jnp.float32),
        pltpu.VMEM((N_NODES,), jnp.float32),
        pltpu.VMEM((N_NODES,), jnp.float32),
        pltpu.VMEM((N_NODES,), jnp.float32),
        pltpu.VMEM((N_NODES,), jnp.float32),
        pltpu.VMEM((N_NODES,), jnp.float32),
        pltpu.VMEM((N_NODES,), jnp.float32),
        pltpu.VMEM((2 * SLOT,), jnp.int32),
        pltpu.VMEM((2 * SLOT,), jnp.int32),
        pltpu.SemaphoreType.DMA,
        pltpu.SemaphoreType.DMA,
        pltpu.SemaphoreType.DMA,
        pltpu.SemaphoreType.DMA,
    ],
)(_segmax_body)


def kernel(node_features, senders, receivers, W1, b1, W2, b2):
    h_T = pl.pallas_call(
        _mm1_body,
        out_shape=jax.ShapeDtypeStruct((D, N_NODES), jnp.float32),
    )(W1, node_features, b1.reshape(D, 1))

    agg_flat = _segmax(h_T.reshape(-1),
                       senders.astype(jnp.int32),
                       receivers.astype(jnp.int32))

    nodes = pl.pallas_call(
        _mm2_body,
        out_shape=jax.ShapeDtypeStruct((N_NODES, D), jnp.float32),
    )(agg_flat.reshape(D, N_NODES), W2, b2.reshape(1, D))
    return nodes
